# baseline (speedup 1.0000x reference)
import functools

import jax
import jax.numpy as jnp
from jax import lax
from jax.experimental import pallas as pl
from jax.experimental.pallas import tpu as pltpu
from jax.experimental.pallas import tpu_sc as plsc


def _sc_dims():
    try:
        info = plsc.get_sparse_core_info()
        return info.num_cores, info.num_subcores
    except Exception:
        return 2, 16


_NC, _NS = _sc_dims()
_NW = _NC * _NS


def _mm_body(x_ref, w_ref, o_ref):
    o_ref[...] = lax.dot_general(
        x_ref[...], w_ref[...],
        (((1,), (1,)), ((), ())),
        preferred_element_type=jnp.float32,
    )


def _down_project(x, w):
    V, H = x.shape
    R = w.shape[0]
    BM = 800
    assert V % BM == 0
    return pl.pallas_call(
        _mm_body,
        grid=(V // BM,),
        in_specs=[
            pl.BlockSpec((BM, H), lambda i: (i, 0)),
            pl.BlockSpec((R, H), lambda i: (0, 0)),
        ],
        out_specs=pl.BlockSpec((BM, R), lambda i: (i, 0)),
        out_shape=jax.ShapeDtypeStruct((V, R), jnp.float32),
        compiler_params=pltpu.CompilerParams(
            dimension_semantics=("arbitrary",),
        ),
    )(x, w)


def _flat_indices(kgl_ids, S):
    B = kgl_ids.shape[0]

    def body(ids_ref, o_ref):
        ids = ids_ref[...]
        o_ref[...] = ids * S + lax.broadcasted_iota(jnp.int32, (B, S), 1)

    out = pl.pallas_call(
        body,
        out_shape=jax.ShapeDtypeStruct((B, S), jnp.int32),
    )(kgl_ids.reshape(B, 1))
    return out.reshape(B * S)


def _sc_tok_mask(gidx, kgl2token_flat):
    N = gidx.shape[0]
    npw = N // _NW
    mesh = plsc.VectorSubcoreMesh(core_axis_name="c", subcore_axis_name="s")

    @functools.partial(
        pl.kernel,
        mesh=mesh,
        out_type=(
            jax.ShapeDtypeStruct((N,), jnp.int32),
            jax.ShapeDtypeStruct((N,), jnp.float32),
        ),
        scratch_types=[
            pltpu.VMEM((npw,), jnp.int32),
            pltpu.VMEM((npw,), jnp.int32),
            pltpu.VMEM((npw,), jnp.float32),
            pltpu.SemaphoreType.DMA,
        ],
        compiler_params=pltpu.CompilerParams(use_tc_tiling_on_sc=False),
    )
    def k(gidx_hbm, tblf_hbm, tok_hbm, maskf_hbm,
          gidx_v, tok_v, mskf_v, sem):
        wid = lax.axis_index("s") * _NC + lax.axis_index("c")
        nbase = wid * npw
        pltpu.sync_copy(gidx_hbm.at[pl.ds(nbase, npw)], gidx_v)
        pltpu.async_copy(tblf_hbm.at[gidx_v], tok_v, sem).wait()

        @pl.loop(0, npw // 16)
        def _(r):
            off = pl.multiple_of(r * 16, 16)
            v = tok_v[pl.ds(off, 16)]
            mskf_v[pl.ds(off, 16)] = jnp.where(
                v > 0, jnp.float32(1.0), jnp.float32(0.0))

        pltpu.sync_copy(tok_v, tok_hbm.at[pl.ds(nbase, npw)])
        pltpu.sync_copy(mskf_v, maskf_hbm.at[pl.ds(nbase, npw)])

    return k(gidx, kgl2token_flat)


def _sc_row_gather(tok, rmat):
    N = tok.shape[0]
    R = rmat.shape[1]
    npw = N // _NW
    CH = 256
    assert npw % CH == 0
    n_ch = npw // CH
    mesh = plsc.VectorSubcoreMesh(core_axis_name="c", subcore_axis_name="s")

    @functools.partial(
        pl.kernel,
        mesh=mesh,
        out_type=jax.ShapeDtypeStruct((N, R), jnp.float32),
        scratch_types=[
            pltpu.VMEM((npw,), jnp.int32),
            pltpu.VMEM((2, CH, R), jnp.float32),
            pltpu.SemaphoreType.DMA,
            pltpu.SemaphoreType.DMA((2,)),
            pltpu.SemaphoreType.DMA((2,)),
        ],
        compiler_params=pltpu.CompilerParams(use_tc_tiling_on_sc=False),
    )
    def k(tok_hbm, rmat_hbm, embs_hbm, tok_v, rows_v, sem, gsem, wsem):
        wid = lax.axis_index("s") * _NC + lax.axis_index("c")
        nbase = wid * npw
        pltpu.sync_copy(tok_hbm.at[pl.ds(nbase, npw)], tok_v)
        gh = [None, None]
        wh = [None, None]
        for c in range(n_ch):
            cur, nxt = c % 2, (c + 1) % 2
            if c == 0:
                gh[0] = pltpu.async_copy(
                    rmat_hbm.at[tok_v.at[pl.ds(0, CH)]],
                    rows_v.at[0], gsem.at[0])
            if c + 1 < n_ch:
                if wh[nxt] is not None:
                    wh[nxt].wait()
                gh[nxt] = pltpu.async_copy(
                    rmat_hbm.at[tok_v.at[pl.ds((c + 1) * CH, CH)]],
                    rows_v.at[nxt], gsem.at[nxt])
            gh[cur].wait()
            wh[cur] = pltpu.async_copy(
                rows_v.at[cur], embs_hbm.at[pl.ds(nbase + c * CH, CH)],
                wsem.at[cur])
        wh[0].wait()
        wh[1].wait()

    return k(tok, rmat)


def _pool_body(mask_ref, maskf_ref, embs_ref, w0_ref, w1_ref, w2_ref, b_ref,
               o_ref, gmean_ref):
    i = pl.program_id(0)

    @pl.when(i == 0)
    def _():
        mf = maskf_ref[...]
        lens = jnp.sum(mf, axis=2)
        gmean_ref[0] = jnp.sum(jnp.log(lens)) / lens.size

    mask = mask_ref[0]
    b3 = mask[:, :, None] > 0.0
    e = embs_ref[...]
    em = jnp.where(b3, e, 0.0)
    s = jnp.sum(em, axis=1)
    sq = jnp.sum(em * em, axis=1)
    mx = jnp.max(jnp.where(b3, e, -1e10), axis=1)
    mn = jnp.min(jnp.where(b3, e, 1e10), axis=1)
    ln = jnp.sum(mask, axis=1, keepdims=True)
    mean = s / (ln + 1e-10)
    sqm = sq / (ln + 1e-10)
    std = jnp.sqrt(jnp.clip(sqm - mean * mean, 1e-6, None))
    feats = jnp.concatenate([mean, mx, mn, std], axis=-1)
    g = gmean_ref[0]
    scale = jnp.log(ln) / (g + 1e-10)
    sinv = 1.0 / jnp.clip(scale, 0.01, None)
    dn = (((1,), (1,)), ((), ()))
    r0 = lax.dot_general(feats, w0_ref[...], dn,
                         preferred_element_type=jnp.float32)
    r1 = lax.dot_general(feats, w1_ref[...], dn,
                         preferred_element_type=jnp.float32)
    r2 = lax.dot_general(feats, w2_ref[...], dn,
                         preferred_element_type=jnp.float32)
    res = r0 + scale * r1 + sinv * r2 + b_ref[...]
    nrm = jnp.sqrt(jnp.sum(res * res, axis=1, keepdims=True))
    o_ref[...] = res / jnp.clip(nrm, 1e-12, None)


def _pool(mask3, mask3_full, embs, w0, w1, w2, b2):
    NB, BB, S = mask3.shape
    NBF = mask3_full.shape[0]
    R = embs.shape[2]
    return pl.pallas_call(
        _pool_body,
        grid=(NB,),
        in_specs=[
            pl.BlockSpec((1, BB, S), lambda i: (i, 0, 0)),
            pl.BlockSpec((NBF, BB, S), lambda i: (0, 0, 0)),
            pl.BlockSpec((BB, S, R), lambda i: (i, 0, 0)),
            pl.BlockSpec((R, 4 * R), lambda i: (0, 0)),
            pl.BlockSpec((R, 4 * R), lambda i: (0, 0)),
            pl.BlockSpec((R, 4 * R), lambda i: (0, 0)),
            pl.BlockSpec((1, R), lambda i: (0, 0)),
        ],
        out_specs=pl.BlockSpec((BB, R), lambda i: (i, 0)),
        out_shape=jax.ShapeDtypeStruct((NB * BB, R), jnp.float32),
        scratch_shapes=[pltpu.SMEM((1,), jnp.float32)],
        compiler_params=pltpu.CompilerParams(
            dimension_semantics=("arbitrary",),
        ),
    )(mask3, mask3_full, embs, w0, w1, w2, b2)


def kernel(kgl_ids, kgl2token, text_embeddings, W_down, W_re, b_re):
    B = kgl_ids.shape[0]
    S = kgl2token.shape[1]
    R = W_down.shape[0]
    rmat = _down_project(text_embeddings, W_down)
    gidx = _flat_indices(kgl_ids, S)
    tok, maskf = _sc_tok_mask(gidx, kgl2token.reshape(-1))
    N = B * S
    H2 = N // 2
    embs0 = _sc_row_gather(tok[:H2], rmat)
    embs1 = _sc_row_gather(tok[H2:], rmat)
    BB = B // _NW
    mask3 = maskf.reshape(_NW, BB, S)
    w0 = W_re[:, 0::3]
    w1 = W_re[:, 1::3]
    w2 = W_re[:, 2::3]
    b2 = b_re.reshape(1, R)
    NH = _NW // 2
    out0 = _pool(mask3[:NH], mask3, embs0.reshape(B // 2, S, R),
                 w0, w1, w2, b2)
    out1 = _pool(mask3[NH:], mask3, embs1.reshape(B // 2, S, R),
                 w0, w1, w2, b2)
    return jnp.concatenate([out0, out1], axis=0)


# speedup vs baseline: 1.3382x; 1.3382x over previous
import functools

import jax
import jax.numpy as jnp
from jax import lax
from jax.experimental import pallas as pl
from jax.experimental.pallas import tpu as pltpu
from jax.experimental.pallas import tpu_sc as plsc


def _sc_dims():
    try:
        info = plsc.get_sparse_core_info()
        return info.num_cores, info.num_subcores
    except Exception:
        return 2, 16


_NC, _NS = _sc_dims()
_NW = _NC * _NS


def _mm_body(x_ref, w_ref, o_ref):
    o_ref[...] = lax.dot_general(
        x_ref[...], w_ref[...],
        (((1,), (1,)), ((), ())),
        preferred_element_type=jnp.float32,
    )


def _down_project(x, w):
    V, H = x.shape
    R = w.shape[0]
    BM = 1000
    assert V % BM == 0
    return pl.pallas_call(
        _mm_body,
        grid=(V // BM,),
        in_specs=[
            pl.BlockSpec((BM, H), lambda i: (i, 0)),
            pl.BlockSpec((R, H), lambda i: (0, 0)),
        ],
        out_specs=pl.BlockSpec((BM, R), lambda i: (i, 0)),
        out_shape=jax.ShapeDtypeStruct((V, R), jnp.float32),
        compiler_params=pltpu.CompilerParams(
            dimension_semantics=("arbitrary",),
        ),
    )(x, w)


def _flat_indices(kgl_ids, S):
    B = kgl_ids.shape[0]

    def body(ids_ref, o_ref):
        ids = ids_ref[...]
        o_ref[...] = ids * S + lax.broadcasted_iota(jnp.int32, (B, S), 1)

    out = pl.pallas_call(
        body,
        out_shape=jax.ShapeDtypeStruct((B, S), jnp.int32),
    )(kgl_ids.reshape(B, 1))
    return out.reshape(B * S)


def _sc_tok_mask(gidx, kgl2token_flat):
    N = gidx.shape[0]
    npw = N // _NW
    mesh = plsc.VectorSubcoreMesh(core_axis_name="c", subcore_axis_name="s")

    @functools.partial(
        pl.kernel,
        mesh=mesh,
        out_type=(
            jax.ShapeDtypeStruct((N,), jnp.int32),
            jax.ShapeDtypeStruct((N,), jnp.float32),
        ),
        scratch_types=[
            pltpu.VMEM((npw,), jnp.int32),
            pltpu.VMEM((npw,), jnp.int32),
            pltpu.VMEM((npw,), jnp.float32),
            pltpu.SemaphoreType.DMA,
        ],
        compiler_params=pltpu.CompilerParams(use_tc_tiling_on_sc=False),
    )
    def k(gidx_hbm, tblf_hbm, tok_hbm, maskf_hbm,
          gidx_v, tok_v, mskf_v, sem):
        wid = lax.axis_index("s") * _NC + lax.axis_index("c")
        nbase = wid * npw
        pltpu.sync_copy(gidx_hbm.at[pl.ds(nbase, npw)], gidx_v)
        pltpu.async_copy(tblf_hbm.at[gidx_v], tok_v, sem).wait()

        @pl.loop(0, npw // 16)
        def _(r):
            off = pl.multiple_of(r * 16, 16)
            v = tok_v[pl.ds(off, 16)]
            mskf_v[pl.ds(off, 16)] = jnp.where(
                v > 0, jnp.float32(1.0), jnp.float32(0.0))

        pltpu.sync_copy(tok_v, tok_hbm.at[pl.ds(nbase, npw)])
        pltpu.sync_copy(mskf_v, maskf_hbm.at[pl.ds(nbase, npw)])

    return k(gidx, kgl2token_flat)


def _sc_row_gather(tok, rmat):
    N = tok.shape[0]
    R = rmat.shape[1]
    npw = N // _NW
    CH = 256
    assert npw % CH == 0
    n_ch = npw // CH
    mesh = plsc.VectorSubcoreMesh(core_axis_name="c", subcore_axis_name="s")

    @functools.partial(
        pl.kernel,
        mesh=mesh,
        out_type=jax.ShapeDtypeStruct((N, R), jnp.float32),
        scratch_types=[
            pltpu.VMEM((npw,), jnp.int32),
            pltpu.VMEM((2, CH, R), jnp.float32),
            pltpu.SemaphoreType.DMA,
            pltpu.SemaphoreType.DMA((2,)),
            pltpu.SemaphoreType.DMA((2,)),
        ],
        compiler_params=pltpu.CompilerParams(use_tc_tiling_on_sc=False),
    )
    def k(tok_hbm, rmat_hbm, embs_hbm, tok_v, rows_v, sem, gsem, wsem):
        wid = lax.axis_index("s") * _NC + lax.axis_index("c")
        nbase = wid * npw
        pltpu.sync_copy(tok_hbm.at[pl.ds(nbase, npw)], tok_v)
        gh = [None, None]
        wh = [None, None]
        for c in range(n_ch):
            cur, nxt = c % 2, (c + 1) % 2
            if c == 0:
                gh[0] = pltpu.async_copy(
                    rmat_hbm.at[tok_v.at[pl.ds(0, CH)]],
                    rows_v.at[0], gsem.at[0])
            if c + 1 < n_ch:
                if wh[nxt] is not None:
                    wh[nxt].wait()
                gh[nxt] = pltpu.async_copy(
                    rmat_hbm.at[tok_v.at[pl.ds((c + 1) * CH, CH)]],
                    rows_v.at[nxt], gsem.at[nxt])
            gh[cur].wait()
            wh[cur] = pltpu.async_copy(
                rows_v.at[cur], embs_hbm.at[pl.ds(nbase + c * CH, CH)],
                wsem.at[cur])
        wh[0].wait()
        wh[1].wait()

    return k(tok, rmat)


def _pool_body(mask_ref, maskf_ref, embs_ref, w0_ref, w1_ref, w2_ref, b_ref,
               o_ref, gmean_ref):
    i = pl.program_id(0)

    @pl.when(i == 0)
    def _():
        mf = maskf_ref[...]
        lens = jnp.sum(mf, axis=2)
        gmean_ref[0] = jnp.sum(jnp.log(lens)) / lens.size

    mask = mask_ref[0]
    b3 = mask[:, :, None] > 0.0
    BB, S = mask.shape
    e2 = embs_ref[...]
    e = e2.reshape(BB, S, e2.shape[1])
    em = jnp.where(b3, e, 0.0)
    s = jnp.sum(em, axis=1)
    sq = jnp.sum(em * em, axis=1)
    mx = jnp.max(jnp.where(b3, e, -1e10), axis=1)
    mn = jnp.min(jnp.where(b3, e, 1e10), axis=1)
    ln = jnp.sum(mask, axis=1, keepdims=True)
    mean = s / (ln + 1e-10)
    sqm = sq / (ln + 1e-10)
    std = jnp.sqrt(jnp.clip(sqm - mean * mean, 1e-6, None))
    feats = jnp.concatenate([mean, mx, mn, std], axis=-1)
    g = gmean_ref[0]
    scale = jnp.log(ln) / (g + 1e-10)
    sinv = 1.0 / jnp.clip(scale, 0.01, None)
    dn = (((1,), (1,)), ((), ()))
    r0 = lax.dot_general(feats, w0_ref[...], dn,
                         preferred_element_type=jnp.float32)
    r1 = lax.dot_general(feats, w1_ref[...], dn,
                         preferred_element_type=jnp.float32)
    r2 = lax.dot_general(feats, w2_ref[...], dn,
                         preferred_element_type=jnp.float32)
    res = r0 + scale * r1 + sinv * r2 + b_ref[...]
    nrm = jnp.sqrt(jnp.sum(res * res, axis=1, keepdims=True))
    o_ref[...] = res / jnp.clip(nrm, 1e-12, None)


def _pool(mask3, mask3_full, embs, w0, w1, w2, b2):
    NB, BB, S = mask3.shape
    NBF = mask3_full.shape[0]
    R = embs.shape[1]
    return pl.pallas_call(
        _pool_body,
        grid=(NB,),
        in_specs=[
            pl.BlockSpec((1, BB, S), lambda i: (i, 0, 0)),
            pl.BlockSpec((NBF, BB, S), lambda i: (0, 0, 0)),
            pl.BlockSpec((BB * S, R), lambda i: (i, 0)),
            pl.BlockSpec((R, 4 * R), lambda i: (0, 0)),
            pl.BlockSpec((R, 4 * R), lambda i: (0, 0)),
            pl.BlockSpec((R, 4 * R), lambda i: (0, 0)),
            pl.BlockSpec((1, R), lambda i: (0, 0)),
        ],
        out_specs=pl.BlockSpec((BB, R), lambda i: (i, 0)),
        out_shape=jax.ShapeDtypeStruct((NB * BB, R), jnp.float32),
        scratch_shapes=[pltpu.SMEM((1,), jnp.float32)],
        compiler_params=pltpu.CompilerParams(
            dimension_semantics=("arbitrary",),
        ),
    )(mask3, mask3_full, embs, w0, w1, w2, b2)


def kernel(kgl_ids, kgl2token, text_embeddings, W_down, W_re, b_re):
    B = kgl_ids.shape[0]
    S = kgl2token.shape[1]
    R = W_down.shape[0]
    rmat = _down_project(text_embeddings, W_down)
    gidx = _flat_indices(kgl_ids, S)
    tok, maskf = _sc_tok_mask(gidx, kgl2token.reshape(-1))
    N = B * S
    H2 = N // 2
    embs0 = _sc_row_gather(tok[:H2], rmat)
    embs1 = _sc_row_gather(tok[H2:], rmat)
    BB = B // _NW
    mask3 = maskf.reshape(_NW, BB, S)
    w0 = W_re[:, 0::3]
    w1 = W_re[:, 1::3]
    w2 = W_re[:, 2::3]
    b2 = b_re.reshape(1, R)
    NH = _NW // 2
    out0 = _pool(mask3[:NH], mask3, embs0, w0, w1, w2, b2)
    out1 = _pool(mask3[NH:], mask3, embs1, w0, w1, w2, b2)
    return jnp.concatenate([out0, out1], axis=0)


# speedup vs baseline: 1.3393x; 1.0008x over previous
import functools

import jax
import jax.numpy as jnp
from jax import lax
from jax.experimental import pallas as pl
from jax.experimental.pallas import tpu as pltpu
from jax.experimental.pallas import tpu_sc as plsc


def _sc_dims():
    try:
        info = plsc.get_sparse_core_info()
        return info.num_cores, info.num_subcores
    except Exception:
        return 2, 16


_NC, _NS = _sc_dims()
_NW = _NC * _NS


def _mm_body(x0_ref, x1_ref, w0_ref, w1_ref, o_ref):
    dn = (((1,), (1,)), ((), ()))
    o_ref[...] = (
        lax.dot_general(x0_ref[...], w0_ref[...], dn,
                        preferred_element_type=jnp.float32)
        + lax.dot_general(x1_ref[...], w1_ref[...], dn,
                          preferred_element_type=jnp.float32)
    )


def _down_project(x, w):
    V, H = x.shape
    R = w.shape[0]
    BM = 1000
    H2 = H // 2
    assert V % BM == 0
    return pl.pallas_call(
        _mm_body,
        grid=(V // BM,),
        in_specs=[
            pl.BlockSpec((BM, H2), lambda i: (i, 0)),
            pl.BlockSpec((BM, H2), lambda i: (i, 1)),
            pl.BlockSpec((R, H2), lambda i: (0, 0)),
            pl.BlockSpec((R, H2), lambda i: (0, 1)),
        ],
        out_specs=pl.BlockSpec((BM, R), lambda i: (i, 0)),
        out_shape=jax.ShapeDtypeStruct((V, R), jnp.float32),
        compiler_params=pltpu.CompilerParams(
            dimension_semantics=("arbitrary",),
        ),
    )(x, x, w, w)


def _flat_indices(kgl_ids, S):
    B = kgl_ids.shape[0]

    def body(ids_ref, o_ref):
        ids = ids_ref[...]
        o_ref[...] = ids * S + lax.broadcasted_iota(jnp.int32, (B, S), 1)

    out = pl.pallas_call(
        body,
        out_shape=jax.ShapeDtypeStruct((B, S), jnp.int32),
    )(kgl_ids.reshape(B, 1))
    return out.reshape(B * S)


def _sc_tok_mask(gidx, kgl2token_flat):
    N = gidx.shape[0]
    npw = N // _NW
    mesh = plsc.VectorSubcoreMesh(core_axis_name="c", subcore_axis_name="s")

    @functools.partial(
        pl.kernel,
        mesh=mesh,
        out_type=(
            jax.ShapeDtypeStruct((N,), jnp.int32),
            jax.ShapeDtypeStruct((N,), jnp.float32),
        ),
        scratch_types=[
            pltpu.VMEM((npw,), jnp.int32),
            pltpu.VMEM((npw,), jnp.int32),
            pltpu.VMEM((npw,), jnp.float32),
            pltpu.SemaphoreType.DMA,
        ],
        compiler_params=pltpu.CompilerParams(use_tc_tiling_on_sc=False),
    )
    def k(gidx_hbm, tblf_hbm, tok_hbm, maskf_hbm,
          gidx_v, tok_v, mskf_v, sem):
        wid = lax.axis_index("s") * _NC + lax.axis_index("c")
        nbase = wid * npw
        pltpu.sync_copy(gidx_hbm.at[pl.ds(nbase, npw)], gidx_v)
        pltpu.async_copy(tblf_hbm.at[gidx_v], tok_v, sem).wait()

        @pl.loop(0, npw // 16)
        def _(r):
            off = pl.multiple_of(r * 16, 16)
            v = tok_v[pl.ds(off, 16)]
            mskf_v[pl.ds(off, 16)] = jnp.where(
                v > 0, jnp.float32(1.0), jnp.float32(0.0))

        pltpu.sync_copy(tok_v, tok_hbm.at[pl.ds(nbase, npw)])
        pltpu.sync_copy(mskf_v, maskf_hbm.at[pl.ds(nbase, npw)])

    return k(gidx, kgl2token_flat)


def _sc_row_gather(tok, rmat):
    N = tok.shape[0]
    R = rmat.shape[1]
    npw = N // _NW
    CH = 256
    assert npw % CH == 0
    n_ch = npw // CH
    mesh = plsc.VectorSubcoreMesh(core_axis_name="c", subcore_axis_name="s")

    @functools.partial(
        pl.kernel,
        mesh=mesh,
        out_type=jax.ShapeDtypeStruct((N, R), jnp.float32),
        scratch_types=[
            pltpu.VMEM((npw,), jnp.int32),
            pltpu.VMEM((2, CH, R), jnp.float32),
            pltpu.SemaphoreType.DMA,
            pltpu.SemaphoreType.DMA((2,)),
            pltpu.SemaphoreType.DMA((2,)),
        ],
        compiler_params=pltpu.CompilerParams(use_tc_tiling_on_sc=False),
    )
    def k(tok_hbm, rmat_hbm, embs_hbm, tok_v, rows_v, sem, gsem, wsem):
        wid = lax.axis_index("s") * _NC + lax.axis_index("c")
        nbase = wid * npw
        pltpu.sync_copy(tok_hbm.at[pl.ds(nbase, npw)], tok_v)
        gh = [None, None]
        wh = [None, None]
        for c in range(n_ch):
            cur, nxt = c % 2, (c + 1) % 2
            if c == 0:
                gh[0] = pltpu.async_copy(
                    rmat_hbm.at[tok_v.at[pl.ds(0, CH)]],
                    rows_v.at[0], gsem.at[0])
            if c + 1 < n_ch:
                if wh[nxt] is not None:
                    wh[nxt].wait()
                gh[nxt] = pltpu.async_copy(
                    rmat_hbm.at[tok_v.at[pl.ds((c + 1) * CH, CH)]],
                    rows_v.at[nxt], gsem.at[nxt])
            gh[cur].wait()
            wh[cur] = pltpu.async_copy(
                rows_v.at[cur], embs_hbm.at[pl.ds(nbase + c * CH, CH)],
                wsem.at[cur])
        wh[0].wait()
        wh[1].wait()

    return k(tok, rmat)


def _pool_body(mask_ref, maskf_ref, embs_ref, w0_ref, w1_ref, w2_ref, b_ref,
               o_ref, gmean_ref):
    i = pl.program_id(0)

    @pl.when(i == 0)
    def _():
        mf = maskf_ref[...]
        lens = jnp.sum(mf, axis=2)
        gmean_ref[0] = jnp.sum(jnp.log(lens)) / lens.size

    mask = mask_ref[0]
    b3 = mask[:, :, None] > 0.0
    BB, S = mask.shape
    e2 = embs_ref[...]
    e = e2.reshape(BB, S, e2.shape[1])
    em = jnp.where(b3, e, 0.0)
    s = jnp.sum(em, axis=1)
    sq = jnp.sum(em * em, axis=1)
    mx = jnp.max(jnp.where(b3, e, -1e10), axis=1)
    mn = jnp.min(jnp.where(b3, e, 1e10), axis=1)
    ln = jnp.sum(mask, axis=1, keepdims=True)
    mean = s / (ln + 1e-10)
    sqm = sq / (ln + 1e-10)
    std = jnp.sqrt(jnp.clip(sqm - mean * mean, 1e-6, None))
    feats = jnp.concatenate([mean, mx, mn, std], axis=-1)
    g = gmean_ref[0]
    scale = jnp.log(ln) / (g + 1e-10)
    sinv = 1.0 / jnp.clip(scale, 0.01, None)
    dn = (((1,), (1,)), ((), ()))
    r0 = lax.dot_general(feats, w0_ref[...], dn,
                         preferred_element_type=jnp.float32)
    r1 = lax.dot_general(feats, w1_ref[...], dn,
                         preferred_element_type=jnp.float32)
    r2 = lax.dot_general(feats, w2_ref[...], dn,
                         preferred_element_type=jnp.float32)
    res = r0 + scale * r1 + sinv * r2 + b_ref[...]
    nrm = jnp.sqrt(jnp.sum(res * res, axis=1, keepdims=True))
    o_ref[...] = res / jnp.clip(nrm, 1e-12, None)


def _pool(mask3, mask3_full, embs, w0, w1, w2, b2):
    NB, BB, S = mask3.shape
    NBF = mask3_full.shape[0]
    R = embs.shape[1]
    return pl.pallas_call(
        _pool_body,
        grid=(NB,),
        in_specs=[
            pl.BlockSpec((1, BB, S), lambda i: (i, 0, 0)),
            pl.BlockSpec((NBF, BB, S), lambda i: (0, 0, 0)),
            pl.BlockSpec((BB * S, R), lambda i: (i, 0)),
            pl.BlockSpec((R, 4 * R), lambda i: (0, 0)),
            pl.BlockSpec((R, 4 * R), lambda i: (0, 0)),
            pl.BlockSpec((R, 4 * R), lambda i: (0, 0)),
            pl.BlockSpec((1, R), lambda i: (0, 0)),
        ],
        out_specs=pl.BlockSpec((BB, R), lambda i: (i, 0)),
        out_shape=jax.ShapeDtypeStruct((NB * BB, R), jnp.float32),
        scratch_shapes=[pltpu.SMEM((1,), jnp.float32)],
        compiler_params=pltpu.CompilerParams(
            dimension_semantics=("arbitrary",),
        ),
    )(mask3, mask3_full, embs, w0, w1, w2, b2)


def kernel(kgl_ids, kgl2token, text_embeddings, W_down, W_re, b_re):
    B = kgl_ids.shape[0]
    S = kgl2token.shape[1]
    R = W_down.shape[0]
    rmat = _down_project(text_embeddings, W_down)
    gidx = _flat_indices(kgl_ids, S)
    tok, maskf = _sc_tok_mask(gidx, kgl2token.reshape(-1))
    N = B * S
    H2 = N // 2
    embs0 = _sc_row_gather(tok[:H2], rmat)
    embs1 = _sc_row_gather(tok[H2:], rmat)
    BB = B // _NW
    mask3 = maskf.reshape(_NW, BB, S)
    w0 = W_re[:, 0::3]
    w1 = W_re[:, 1::3]
    w2 = W_re[:, 2::3]
    b2 = b_re.reshape(1, R)
    NH = _NW // 2
    out0 = _pool(mask3[:NH], mask3, embs0, w0, w1, w2, b2)
    out1 = _pool(mask3[NH:], mask3, embs1, w0, w1, w2, b2)
    return jnp.concatenate([out0, out1], axis=0)


# speedup vs baseline: 1.3415x; 1.0016x over previous
import functools

import jax
import jax.numpy as jnp
from jax import lax
from jax.experimental import pallas as pl
from jax.experimental.pallas import tpu as pltpu
from jax.experimental.pallas import tpu_sc as plsc


def _sc_dims():
    try:
        info = plsc.get_sparse_core_info()
        return info.num_cores, info.num_subcores
    except Exception:
        return 2, 16


_NC, _NS = _sc_dims()
_NW = _NC * _NS


def _mm_body(x_ref, w_ref, o_ref):
    o_ref[...] = lax.dot_general(
        x_ref[...], w_ref[...],
        (((1,), (1,)), ((), ())),
        preferred_element_type=jnp.float32,
    )


def _down_project(x, w):
    V, H = x.shape
    R = w.shape[0]
    BM = 1000
    assert V % BM == 0
    return pl.pallas_call(
        _mm_body,
        grid=(V // BM,),
        in_specs=[
            pl.BlockSpec((BM, H), lambda i: (i, 0)),
            pl.BlockSpec((R, H), lambda i: (0, 0)),
        ],
        out_specs=pl.BlockSpec((BM, R), lambda i: (i, 0)),
        out_shape=jax.ShapeDtypeStruct((V, R), jnp.float32),
        compiler_params=pltpu.CompilerParams(
            dimension_semantics=("arbitrary",),
        ),
    )(x, w)


def _flat_indices(kgl_ids, S):
    B = kgl_ids.shape[0]

    def body(ids_ref, o_ref):
        ids = ids_ref[...]
        o_ref[...] = ids * S + lax.broadcasted_iota(jnp.int32, (B, S), 1)

    out = pl.pallas_call(
        body,
        out_shape=jax.ShapeDtypeStruct((B, S), jnp.int32),
    )(kgl_ids.reshape(B, 1))
    return out.reshape(B * S)


def _sc_tok_mask(gidx, kgl2token_flat):
    N = gidx.shape[0]
    npw = N // _NW
    mesh = plsc.VectorSubcoreMesh(core_axis_name="c", subcore_axis_name="s")

    @functools.partial(
        pl.kernel,
        mesh=mesh,
        out_type=(
            jax.ShapeDtypeStruct((N,), jnp.int32),
            jax.ShapeDtypeStruct((N,), jnp.float32),
        ),
        scratch_types=[
            pltpu.VMEM((npw,), jnp.int32),
            pltpu.VMEM((npw,), jnp.int32),
            pltpu.VMEM((npw,), jnp.float32),
            pltpu.SemaphoreType.DMA,
        ],
        compiler_params=pltpu.CompilerParams(use_tc_tiling_on_sc=False),
    )
    def k(gidx_hbm, tblf_hbm, tok_hbm, maskf_hbm,
          gidx_v, tok_v, mskf_v, sem):
        wid = lax.axis_index("s") * _NC + lax.axis_index("c")
        nbase = wid * npw
        pltpu.sync_copy(gidx_hbm.at[pl.ds(nbase, npw)], gidx_v)
        pltpu.async_copy(tblf_hbm.at[gidx_v], tok_v, sem).wait()

        @pl.loop(0, npw // 16)
        def _(r):
            off = pl.multiple_of(r * 16, 16)
            v = tok_v[pl.ds(off, 16)]
            mskf_v[pl.ds(off, 16)] = jnp.where(
                v > 0, jnp.float32(1.0), jnp.float32(0.0))

        pltpu.sync_copy(tok_v, tok_hbm.at[pl.ds(nbase, npw)])
        pltpu.sync_copy(mskf_v, maskf_hbm.at[pl.ds(nbase, npw)])

    return k(gidx, kgl2token_flat)


def _sc_row_gather(tok, rmat):
    N = tok.shape[0]
    R = rmat.shape[1]
    npw = N // _NW
    CH = 256
    assert npw % CH == 0
    n_ch = npw // CH
    mesh = plsc.VectorSubcoreMesh(core_axis_name="c", subcore_axis_name="s")

    @functools.partial(
        pl.kernel,
        mesh=mesh,
        out_type=jax.ShapeDtypeStruct((N, R), jnp.float32),
        scratch_types=[
            pltpu.VMEM((npw,), jnp.int32),
            pltpu.VMEM((2, CH, R), jnp.float32),
            pltpu.SemaphoreType.DMA,
            pltpu.SemaphoreType.DMA((2,)),
            pltpu.SemaphoreType.DMA((2,)),
        ],
        compiler_params=pltpu.CompilerParams(use_tc_tiling_on_sc=False),
    )
    def k(tok_hbm, rmat_hbm, embs_hbm, tok_v, rows_v, sem, gsem, wsem):
        wid = lax.axis_index("s") * _NC + lax.axis_index("c")
        nbase = wid * npw
        pltpu.sync_copy(tok_hbm.at[pl.ds(nbase, npw)], tok_v)
        gh = [None, None]
        wh = [None, None]
        for c in range(n_ch):
            cur, nxt = c % 2, (c + 1) % 2
            if c == 0:
                gh[0] = pltpu.async_copy(
                    rmat_hbm.at[tok_v.at[pl.ds(0, CH)]],
                    rows_v.at[0], gsem.at[0])
            if c + 1 < n_ch:
                if wh[nxt] is not None:
                    wh[nxt].wait()
                gh[nxt] = pltpu.async_copy(
                    rmat_hbm.at[tok_v.at[pl.ds((c + 1) * CH, CH)]],
                    rows_v.at[nxt], gsem.at[nxt])
            gh[cur].wait()
            wh[cur] = pltpu.async_copy(
                rows_v.at[cur], embs_hbm.at[pl.ds(nbase + c * CH, CH)],
                wsem.at[cur])
        wh[0].wait()
        wh[1].wait()

    return k(tok, rmat)


def _pool_body(mask_ref, maskf_ref, embs_ref, w0_ref, w1_ref, w2_ref, b_ref,
               o_ref, gmean_ref):
    i = pl.program_id(0)

    @pl.when(i == 0)
    def _():
        mf = maskf_ref[...]
        lens = jnp.sum(mf, axis=2)
        gmean_ref[0] = jnp.sum(jnp.log(lens)) / lens.size

    mask = mask_ref[0]
    b3 = mask[:, :, None] > 0.0
    BB, S = mask.shape
    e2 = embs_ref[...]
    e = e2.reshape(BB, S, e2.shape[1])
    em = jnp.where(b3, e, 0.0)
    s = jnp.sum(em, axis=1)
    sq = jnp.sum(em * em, axis=1)
    mx = jnp.max(jnp.where(b3, e, -1e10), axis=1)
    mn = jnp.min(jnp.where(b3, e, 1e10), axis=1)
    ln = jnp.sum(mask, axis=1, keepdims=True)
    mean = s / (ln + 1e-10)
    sqm = sq / (ln + 1e-10)
    std = jnp.sqrt(jnp.clip(sqm - mean * mean, 1e-6, None))
    feats = jnp.concatenate([mean, mx, mn, std], axis=-1)
    g = gmean_ref[0]
    scale = jnp.log(ln) / (g + 1e-10)
    sinv = 1.0 / jnp.clip(scale, 0.01, None)
    dn = (((1,), (1,)), ((), ()))
    r0 = lax.dot_general(feats, w0_ref[...], dn,
                         preferred_element_type=jnp.float32)
    r1 = lax.dot_general(feats, w1_ref[...], dn,
                         preferred_element_type=jnp.float32)
    r2 = lax.dot_general(feats, w2_ref[...], dn,
                         preferred_element_type=jnp.float32)
    res = r0 + scale * r1 + sinv * r2 + b_ref[...]
    nrm = jnp.sqrt(jnp.sum(res * res, axis=1, keepdims=True))
    o_ref[...] = res / jnp.clip(nrm, 1e-12, None)


def _pool(mask3, mask3_full, embs, w0, w1, w2, b2):
    NB, BB, S = mask3.shape
    NBF = mask3_full.shape[0]
    R = embs.shape[1]
    return pl.pallas_call(
        _pool_body,
        grid=(NB,),
        in_specs=[
            pl.BlockSpec((1, BB, S), lambda i: (i, 0, 0)),
            pl.BlockSpec((NBF, BB, S), lambda i: (0, 0, 0)),
            pl.BlockSpec((BB * S, R), lambda i: (i, 0)),
            pl.BlockSpec((R, 4 * R), lambda i: (0, 0)),
            pl.BlockSpec((R, 4 * R), lambda i: (0, 0)),
            pl.BlockSpec((R, 4 * R), lambda i: (0, 0)),
            pl.BlockSpec((1, R), lambda i: (0, 0)),
        ],
        out_specs=pl.BlockSpec((BB, R), lambda i: (i, 0)),
        out_shape=jax.ShapeDtypeStruct((NB * BB, R), jnp.float32),
        scratch_shapes=[pltpu.SMEM((1,), jnp.float32)],
        compiler_params=pltpu.CompilerParams(
            dimension_semantics=("arbitrary",),
        ),
    )(mask3, mask3_full, embs, w0, w1, w2, b2)


def kernel(kgl_ids, kgl2token, text_embeddings, W_down, W_re, b_re):
    B = kgl_ids.shape[0]
    S = kgl2token.shape[1]
    R = W_down.shape[0]
    rmat = _down_project(text_embeddings, W_down)
    gidx = _flat_indices(kgl_ids, S)
    tok, maskf = _sc_tok_mask(gidx, kgl2token.reshape(-1))
    N = B * S
    H2 = N // 2
    embs0 = _sc_row_gather(tok[:H2], rmat)
    embs1 = _sc_row_gather(tok[H2:], rmat)
    BB = B // _NW
    mask3 = maskf.reshape(_NW, BB, S)
    w0 = W_re[:, 0::3]
    w1 = W_re[:, 1::3]
    w2 = W_re[:, 2::3]
    b2 = b_re.reshape(1, R)
    NH = _NW // 2
    out0 = _pool(mask3[:NH], mask3, embs0, w0, w1, w2, b2)
    out1 = _pool(mask3[NH:], mask3, embs1, w0, w1, w2, b2)
    return jnp.concatenate([out0, out1], axis=0)
